# Initial kernel scaffold; baseline (speedup 1.0000x reference)
import jax
import jax.numpy as jnp
from jax.experimental import pallas as pl


def kernel(hidden_states, gate_w, w_gate, w_up, w_down, sh_gate, sh_up, sh_down):
    raise NotImplementedError("write your pallas kernel here")


# speedup vs baseline: 2.2092x; 2.2092x over previous
import jax
import jax.numpy as jnp
from jax.experimental import pallas as pl
from jax.experimental.pallas import tpu as pltpu

B, S, H = 1, 2048, 1024
E = 8
I = 512
ISH = 1024
T = B * S
BT = 512
NTB = T // BT


def _dot_nt(a, b):
    return jax.lax.dot_general(a, b, (((1,), (1,)), ((), ())),
                               preferred_element_type=jnp.float32)


def _moe_kernel(x_ref, gatew_ref, wg_ref, wu_ref, wd_ref,
                shg_ref, shu_ref, shd_ref, out_ref, comb_ref):
    e = pl.program_id(0)

    @pl.when(e == 0)
    def _gate_and_shared():
        for tb in range(NTB):
            sl = pl.ds(tb * BT, BT)
            xb = x_ref[sl, :]
            logits = _dot_nt(xb, gatew_ref[...])
            m = jnp.max(logits, axis=-1, keepdims=True)
            ex = jnp.exp(logits - m)
            scores = ex / jnp.sum(ex, axis=-1, keepdims=True)
            eidx = jax.lax.broadcasted_iota(jnp.int32, (BT, E), 1)
            i1 = jnp.argmax(scores, axis=-1)
            m1 = jnp.max(scores, axis=-1)
            masked = jnp.where(eidx == i1[:, None], -jnp.inf, scores)
            i2 = jnp.argmax(masked, axis=-1)
            m2 = jnp.max(masked, axis=-1)
            denom = m1 + m2 + 1e-20
            w1 = (m1 / denom)[:, None]
            w2 = (m2 / denom)[:, None]
            comb = (jnp.where(eidx == i1[:, None], w1, 0.0)
                    + jnp.where(eidx == i2[:, None], w2, 0.0))
            comb_ref[sl, :] = comb
            xb16 = xb.astype(jnp.bfloat16)
            sg = _dot_nt(xb16, shg_ref[...].astype(jnp.bfloat16))
            su = _dot_nt(xb16, shu_ref[...].astype(jnp.bfloat16))
            act = (sg * jax.lax.logistic(sg)) * su
            ys = _dot_nt(act.astype(jnp.bfloat16),
                         shd_ref[...].astype(jnp.bfloat16))
            out_ref[sl, :] = ys

    @pl.when(e > 0)
    def _routed():
        wg = wg_ref[0].astype(jnp.bfloat16)
        wu = wu_ref[0].astype(jnp.bfloat16)
        wd = wd_ref[0].astype(jnp.bfloat16)
        onehot = (jax.lax.broadcasted_iota(jnp.int32, (E, 1), 0)
                  == e - 1).astype(jnp.float32)
        for tb in range(NTB):
            sl = pl.ds(tb * BT, BT)
            xb16 = x_ref[sl, :].astype(jnp.bfloat16)
            g = _dot_nt(xb16, wg)
            u = _dot_nt(xb16, wu)
            a = (g * jax.lax.logistic(g)) * u
            eo = _dot_nt(a.astype(jnp.bfloat16), wd)
            cw = jnp.dot(comb_ref[sl, :], onehot,
                         preferred_element_type=jnp.float32)
            out_ref[sl, :] += cw * eo


def kernel(hidden_states, gate_w, w_gate, w_up, w_down, sh_gate, sh_up, sh_down):
    x = hidden_states.reshape(T, H)
    out = pl.pallas_call(
        _moe_kernel,
        grid=(E + 1,),
        in_specs=[
            pl.BlockSpec((T, H), lambda e: (0, 0)),
            pl.BlockSpec((E, H), lambda e: (0, 0)),
            pl.BlockSpec((1, I, H), lambda e: (jnp.maximum(e - 1, 0), 0, 0)),
            pl.BlockSpec((1, I, H), lambda e: (jnp.maximum(e - 1, 0), 0, 0)),
            pl.BlockSpec((1, H, I), lambda e: (jnp.maximum(e - 1, 0), 0, 0)),
            pl.BlockSpec((ISH, H), lambda e: (0, 0)),
            pl.BlockSpec((ISH, H), lambda e: (0, 0)),
            pl.BlockSpec((H, ISH), lambda e: (0, 0)),
        ],
        out_specs=pl.BlockSpec((T, H), lambda e: (0, 0)),
        out_shape=jax.ShapeDtypeStruct((T, H), jnp.float32),
        scratch_shapes=[pltpu.VMEM((T, E), jnp.float32)],
    )(x, gate_w, w_gate, w_up, w_down, sh_gate, sh_up, sh_down)
    return out.reshape(hidden_states.shape)
